# baseline (speedup 1.0000x reference)
import jax
import jax.numpy as jnp
from jax import lax
from jax.experimental import pallas as pl

_ROWS_PER_BLOCK = 8
_TOPK = 64


def _count16(pred):
    acc = pred.astype(jnp.int16)
    w = acc.shape[1] // 2
    while w >= 256:
        acc = acc[:, :w] + acc[:, w:]
        w //= 2
    return jnp.sum(acc.astype(jnp.int32), axis=1, keepdims=True)


def _rank_threshold(arr, rank, r, nbits, bias):
    def bit(i, lu):
        b = nbits - 1 - i
        cand_u = lu | (jnp.int32(1) << b)
        cand_s = (cand_u - bias).astype(jnp.int16)
        cnt = jnp.sum((arr >= cand_s).astype(jnp.int32), axis=1,
                      keepdims=True)
        return jnp.where(cnt >= rank, cand_u, lu)

    return lax.fori_loop(0, nbits, bit, jnp.zeros((r, 1), jnp.int32))


def _topk_mask_kernel(x_ref, o_ref):
    x = x_ref[...]
    r, n = x.shape
    y = jnp.maximum(x, 0.0)
    skey = lax.bitcast_convert_type(y, jnp.int32)
    hi = jnp.right_shift(skey, 16).astype(jnp.int16)
    lo = ((skey & jnp.int32(0xFFFF)) - 32768).astype(jnp.int16)

    g = n // 8
    m8f = y[:, 0:g]
    for k in range(1, 8):
        m8f = jnp.maximum(m8f, y[:, k * g:(k + 1) * g])
    m8 = jnp.right_shift(lax.bitcast_convert_type(m8f, jnp.int32),
                         16).astype(jnp.int16)
    hi_lb = _rank_threshold(m8, jnp.int32(_TOPK), r, 15, 0)
    hi_ub = _rank_threshold(m8, jnp.int32(8), r, 15, 0)

    def conv_cond(carry):
        lo_b, hi_b = carry
        return jnp.any(lo_b < hi_b)

    def conv_body(carry):
        lo_b, hi_b = carry
        mid = jnp.right_shift(lo_b + hi_b + 1, 1)
        cnt = _count16(hi >= mid.astype(jnp.int16))
        ge = cnt >= _TOPK
        return jnp.where(ge, mid, lo_b), jnp.where(ge, hi_b, mid - 1)

    t_hi, _ = lax.while_loop(conv_cond, conv_body, (hi_lb, hi_ub))
    th = t_hi.astype(jnp.int16)

    eq_hi = hi == th
    m2 = _TOPK - _count16(hi > th)
    lo_m = jnp.where(eq_hi, lo, jnp.int16(-32768))

    bias = jnp.int32(32768)

    def lo_bit(i, lu):
        bit = 15 - i
        cand_u = lu | (jnp.int32(1) << bit)
        cand_s = (cand_u - bias).astype(jnp.int16)
        cnt = _count16(lo_m >= cand_s)
        return jnp.where(cnt >= m2, cand_u, lu)

    ll = lax.fori_loop(0, 16, lo_bit, jnp.zeros((r, 1), jnp.int32))
    tl = (ll - bias).astype(jnp.int16)

    eq = eq_hi & (lo == tl)
    gt = (hi > th) | (lo_m > tl)
    m = _TOPK - _count16(gt)
    c_eq = _count16(eq)

    h = n // 2
    idx = lax.broadcasted_iota(jnp.int16, (r, h), 1)
    idx = jnp.concatenate([idx, idx + jnp.int16(h)], axis=1)
    idx_m = jnp.where(eq, idx, jnp.int16(32767))

    def tie_search(_):
        def index_bit(i, t):
            bit = 14 - i
            cand = (t | (jnp.int32(1) << bit)).astype(jnp.int16)
            cnt = _count16(idx_m < cand)
            return jnp.where(cnt < m, cand.astype(jnp.int32), t)

        return lax.fori_loop(0, 15, index_bit,
                             jnp.zeros((r, 1), jnp.int32))

    t_idx = lax.cond(jnp.all(m == c_eq),
                     lambda _: jnp.full((r, 1), 32767, jnp.int32),
                     tie_search, 0)
    t16 = t_idx.astype(jnp.int16)

    mask = gt | (eq & (idx_m <= t16))
    o_ref[...] = jnp.where(mask, y, 0.0)


def kernel(x):
    m, n = x.shape
    return pl.pallas_call(
        _topk_mask_kernel,
        grid=(m // _ROWS_PER_BLOCK,),
        in_specs=[pl.BlockSpec((_ROWS_PER_BLOCK, n), lambda i: (i, 0))],
        out_specs=pl.BlockSpec((_ROWS_PER_BLOCK, n), lambda i: (i, 0)),
        out_shape=jax.ShapeDtypeStruct((m, n), x.dtype),
    )(x)


# speedup vs baseline: 1.2371x; 1.2371x over previous
import jax
import jax.numpy as jnp
from jax import lax
from jax.experimental import pallas as pl

_ROWS_PER_BLOCK = 8
_TOPK = 64


def _count16(pred):
    acc = pred.astype(jnp.int16)
    w = acc.shape[1] // 2
    while w >= 256:
        acc = acc[:, :w] + acc[:, w:]
        w //= 2
    return jnp.sum(acc.astype(jnp.int32), axis=1, keepdims=True)


def _topk_mask_kernel(x_ref, o_ref):
    x = x_ref[...]
    r, n = x.shape
    y = jnp.maximum(x, 0.0)
    skey = lax.bitcast_convert_type(y, jnp.int32)
    hi = jnp.right_shift(skey, 16).astype(jnp.int16)
    lo = ((skey & jnp.int32(0xFFFF)) - 32768).astype(jnp.int16)

    def hi_bit(i, lu):
        bit = 14 - i
        cand_u = lu | (jnp.int32(1) << bit)
        cand_s = cand_u.astype(jnp.int16)
        cnt = _count16(hi >= cand_s)
        return jnp.where(cnt >= _TOPK, cand_u, lu)

    lh = lax.fori_loop(0, 15, hi_bit, jnp.zeros((r, 1), jnp.int32))
    th = lh.astype(jnp.int16)

    eq_hi = hi == th
    m2 = _TOPK - _count16(hi > th)
    lo_m = jnp.where(eq_hi, lo, jnp.int16(-32768))

    bias = jnp.int32(32768)

    def lo_bit(i, lu):
        bit = 15 - i
        cand_u = lu | (jnp.int32(1) << bit)
        cand_s = (cand_u - bias).astype(jnp.int16)
        cnt = _count16(lo_m >= cand_s)
        return jnp.where(cnt >= m2, cand_u, lu)

    ll = lax.fori_loop(0, 16, lo_bit, jnp.zeros((r, 1), jnp.int32))
    tl = (ll - bias).astype(jnp.int16)

    eq = eq_hi & (lo == tl)
    gt = (hi > th) | (lo_m > tl)
    m = _TOPK - _count16(gt)
    c_eq = _count16(eq)

    h = n // 2
    idx = lax.broadcasted_iota(jnp.int16, (r, h), 1)
    idx = jnp.concatenate([idx, idx + jnp.int16(h)], axis=1)
    idx_m = jnp.where(eq, idx, jnp.int16(32767))

    def tie_search(_):
        def index_bit(i, t):
            bit = 14 - i
            cand = (t | (jnp.int32(1) << bit)).astype(jnp.int16)
            cnt = _count16(idx_m < cand)
            return jnp.where(cnt < m, cand.astype(jnp.int32), t)

        return lax.fori_loop(0, 15, index_bit,
                             jnp.zeros((r, 1), jnp.int32))

    t_idx = lax.cond(jnp.all(m == c_eq),
                     lambda _: jnp.full((r, 1), 32767, jnp.int32),
                     tie_search, 0)
    t16 = t_idx.astype(jnp.int16)

    mask = gt | (eq & (idx_m <= t16))
    o_ref[...] = jnp.where(mask, y, 0.0)


def kernel(x):
    m, n = x.shape
    return pl.pallas_call(
        _topk_mask_kernel,
        grid=(m // _ROWS_PER_BLOCK,),
        in_specs=[pl.BlockSpec((_ROWS_PER_BLOCK, n), lambda i: (i, 0))],
        out_specs=pl.BlockSpec((_ROWS_PER_BLOCK, n), lambda i: (i, 0)),
        out_shape=jax.ShapeDtypeStruct((m, n), x.dtype),
    )(x)


# speedup vs baseline: 2.2816x; 1.8444x over previous
import jax
import jax.numpy as jnp
from jax import lax
from jax.experimental import pallas as pl

_ROWS_PER_BLOCK = 16
_TOPK = 64


def _count16(pred):
    acc = pred.astype(jnp.int16)
    w = acc.shape[1] // 2
    while w >= 256:
        acc = acc[:, :w] + acc[:, w:]
        w //= 2
    return jnp.sum(acc.astype(jnp.int32), axis=1, keepdims=True)


def _topk_mask_kernel(x_ref, o_ref):
    x = x_ref[...]
    r, n = x.shape
    y = jnp.maximum(x, 0.0)
    skey = lax.bitcast_convert_type(y, jnp.int32)
    hi = jnp.right_shift(skey, 16).astype(jnp.int16)
    lo = ((skey & jnp.int32(0xFFFF)) - 32768).astype(jnp.int16)

    def hi_bit(i, lu):
        bit = 14 - i
        cand_u = lu | (jnp.int32(1) << bit)
        cand_s = cand_u.astype(jnp.int16)
        cnt = _count16(hi >= cand_s)
        return jnp.where(cnt >= _TOPK, cand_u, lu)

    lh = lax.fori_loop(0, 15, hi_bit, jnp.zeros((r, 1), jnp.int32))
    th = lh.astype(jnp.int16)

    eq_hi = hi == th
    m2 = _TOPK - _count16(hi > th)
    lo_m = jnp.where(eq_hi, lo, jnp.int16(-32768))

    bias = jnp.int32(32768)

    def lo_bit(i, lu):
        bit = 15 - i
        cand_u = lu | (jnp.int32(1) << bit)
        cand_s = (cand_u - bias).astype(jnp.int16)
        cnt = _count16(lo_m >= cand_s)
        return jnp.where(cnt >= m2, cand_u, lu)

    ll = lax.fori_loop(0, 16, lo_bit, jnp.zeros((r, 1), jnp.int32))
    tl = (ll - bias).astype(jnp.int16)

    eq = eq_hi & (lo == tl)
    gt = (hi > th) | (lo_m > tl)
    m = _TOPK - _count16(gt)
    c_eq = _count16(eq)

    h = n // 2
    idx = lax.broadcasted_iota(jnp.int16, (r, h), 1)
    idx = jnp.concatenate([idx, idx + jnp.int16(h)], axis=1)
    idx_m = jnp.where(eq, idx, jnp.int16(32767))

    def tie_search(_):
        def index_bit(i, t):
            bit = 14 - i
            cand = (t | (jnp.int32(1) << bit)).astype(jnp.int16)
            cnt = _count16(idx_m < cand)
            return jnp.where(cnt < m, cand.astype(jnp.int32), t)

        return lax.fori_loop(0, 15, index_bit,
                             jnp.zeros((r, 1), jnp.int32))

    t_idx = lax.cond(jnp.all(m == c_eq),
                     lambda _: jnp.full((r, 1), 32767, jnp.int32),
                     tie_search, 0)
    t16 = t_idx.astype(jnp.int16)

    mask = gt | (eq & (idx_m <= t16))
    o_ref[...] = jnp.where(mask, y, 0.0)


def kernel(x):
    m, n = x.shape
    return pl.pallas_call(
        _topk_mask_kernel,
        grid=(m // _ROWS_PER_BLOCK,),
        in_specs=[pl.BlockSpec((_ROWS_PER_BLOCK, n), lambda i: (i, 0))],
        out_specs=pl.BlockSpec((_ROWS_PER_BLOCK, n), lambda i: (i, 0)),
        out_shape=jax.ShapeDtypeStruct((m, n), x.dtype),
    )(x)


# speedup vs baseline: 2.6444x; 1.1590x over previous
import jax
import jax.numpy as jnp
from jax import lax
from jax.experimental import pallas as pl

_ROWS_PER_BLOCK = 32
_TOPK = 64


def _count16(pred):
    acc = pred.astype(jnp.int16)
    w = acc.shape[1] // 2
    while w >= 256:
        acc = acc[:, :w] + acc[:, w:]
        w //= 2
    return jnp.sum(acc.astype(jnp.int32), axis=1, keepdims=True)


def _topk_mask_kernel(x_ref, o_ref):
    x = x_ref[...]
    r, n = x.shape
    y = jnp.maximum(x, 0.0)
    skey = lax.bitcast_convert_type(y, jnp.int32)
    hi = jnp.right_shift(skey, 16).astype(jnp.int16)
    lo = ((skey & jnp.int32(0xFFFF)) - 32768).astype(jnp.int16)

    def hi_bit(i, lu):
        bit = 14 - i
        cand_u = lu | (jnp.int32(1) << bit)
        cand_s = cand_u.astype(jnp.int16)
        cnt = _count16(hi >= cand_s)
        return jnp.where(cnt >= _TOPK, cand_u, lu)

    lh = lax.fori_loop(0, 15, hi_bit, jnp.zeros((r, 1), jnp.int32))
    th = lh.astype(jnp.int16)

    eq_hi = hi == th
    m2 = _TOPK - _count16(hi > th)
    lo_m = jnp.where(eq_hi, lo, jnp.int16(-32768))

    bias = jnp.int32(32768)

    def lo_bit(i, lu):
        bit = 15 - i
        cand_u = lu | (jnp.int32(1) << bit)
        cand_s = (cand_u - bias).astype(jnp.int16)
        cnt = _count16(lo_m >= cand_s)
        return jnp.where(cnt >= m2, cand_u, lu)

    ll = lax.fori_loop(0, 16, lo_bit, jnp.zeros((r, 1), jnp.int32))
    tl = (ll - bias).astype(jnp.int16)

    eq = eq_hi & (lo == tl)
    gt = (hi > th) | (lo_m > tl)
    m = _TOPK - _count16(gt)
    c_eq = _count16(eq)

    h = n // 2
    idx = lax.broadcasted_iota(jnp.int16, (r, h), 1)
    idx = jnp.concatenate([idx, idx + jnp.int16(h)], axis=1)
    idx_m = jnp.where(eq, idx, jnp.int16(32767))

    def tie_search(_):
        def index_bit(i, t):
            bit = 14 - i
            cand = (t | (jnp.int32(1) << bit)).astype(jnp.int16)
            cnt = _count16(idx_m < cand)
            return jnp.where(cnt < m, cand.astype(jnp.int32), t)

        return lax.fori_loop(0, 15, index_bit,
                             jnp.zeros((r, 1), jnp.int32))

    t_idx = lax.cond(jnp.all(m == c_eq),
                     lambda _: jnp.full((r, 1), 32767, jnp.int32),
                     tie_search, 0)
    t16 = t_idx.astype(jnp.int16)

    mask = gt | (eq & (idx_m <= t16))
    o_ref[...] = jnp.where(mask, y, 0.0)


def kernel(x):
    m, n = x.shape
    return pl.pallas_call(
        _topk_mask_kernel,
        grid=(m // _ROWS_PER_BLOCK,),
        in_specs=[pl.BlockSpec((_ROWS_PER_BLOCK, n), lambda i: (i, 0))],
        out_specs=pl.BlockSpec((_ROWS_PER_BLOCK, n), lambda i: (i, 0)),
        out_shape=jax.ShapeDtypeStruct((m, n), x.dtype),
    )(x)


# speedup vs baseline: 2.7897x; 1.0549x over previous
import jax
import jax.numpy as jnp
from jax import lax
from jax.experimental import pallas as pl

_ROWS_PER_BLOCK = 64
_TOPK = 64


def _count16(pred):
    acc = pred.astype(jnp.int16)
    w = acc.shape[1] // 2
    while w >= 256:
        acc = acc[:, :w] + acc[:, w:]
        w //= 2
    return jnp.sum(acc.astype(jnp.int32), axis=1, keepdims=True)


def _topk_mask_kernel(x_ref, o_ref):
    x = x_ref[...]
    r, n = x.shape
    y = jnp.maximum(x, 0.0)
    skey = lax.bitcast_convert_type(y, jnp.int32)
    hi = jnp.right_shift(skey, 16).astype(jnp.int16)
    lo = ((skey & jnp.int32(0xFFFF)) - 32768).astype(jnp.int16)

    def hi_bit(i, lu):
        bit = 14 - i
        cand_u = lu | (jnp.int32(1) << bit)
        cand_s = cand_u.astype(jnp.int16)
        cnt = _count16(hi >= cand_s)
        return jnp.where(cnt >= _TOPK, cand_u, lu)

    lh = lax.fori_loop(0, 15, hi_bit, jnp.zeros((r, 1), jnp.int32))
    th = lh.astype(jnp.int16)

    eq_hi = hi == th
    m2 = _TOPK - _count16(hi > th)
    lo_m = jnp.where(eq_hi, lo, jnp.int16(-32768))

    bias = jnp.int32(32768)

    def lo_bit(i, lu):
        bit = 15 - i
        cand_u = lu | (jnp.int32(1) << bit)
        cand_s = (cand_u - bias).astype(jnp.int16)
        cnt = _count16(lo_m >= cand_s)
        return jnp.where(cnt >= m2, cand_u, lu)

    ll = lax.fori_loop(0, 16, lo_bit, jnp.zeros((r, 1), jnp.int32))
    tl = (ll - bias).astype(jnp.int16)

    eq = eq_hi & (lo == tl)
    gt = (hi > th) | (lo_m > tl)
    m = _TOPK - _count16(gt)
    c_eq = _count16(eq)

    h = n // 2
    idx = lax.broadcasted_iota(jnp.int16, (r, h), 1)
    idx = jnp.concatenate([idx, idx + jnp.int16(h)], axis=1)
    idx_m = jnp.where(eq, idx, jnp.int16(32767))

    def tie_search(_):
        def index_bit(i, t):
            bit = 14 - i
            cand = (t | (jnp.int32(1) << bit)).astype(jnp.int16)
            cnt = _count16(idx_m < cand)
            return jnp.where(cnt < m, cand.astype(jnp.int32), t)

        return lax.fori_loop(0, 15, index_bit,
                             jnp.zeros((r, 1), jnp.int32))

    t_idx = lax.cond(jnp.all(m == c_eq),
                     lambda _: jnp.full((r, 1), 32767, jnp.int32),
                     tie_search, 0)
    t16 = t_idx.astype(jnp.int16)

    mask = gt | (eq & (idx_m <= t16))
    o_ref[...] = jnp.where(mask, y, 0.0)


def kernel(x):
    m, n = x.shape
    return pl.pallas_call(
        _topk_mask_kernel,
        grid=(m // _ROWS_PER_BLOCK,),
        in_specs=[pl.BlockSpec((_ROWS_PER_BLOCK, n), lambda i: (i, 0))],
        out_specs=pl.BlockSpec((_ROWS_PER_BLOCK, n), lambda i: (i, 0)),
        out_shape=jax.ShapeDtypeStruct((m, n), x.dtype),
    )(x)
